# baseline (speedup 1.0000x reference)
import jax
import jax.numpy as jnp
from jax.experimental import pallas as pl
from jax.experimental.pallas import tpu as pltpu

_CAPACITY = 65536
_EMBED = 256
_PTR = 0
_NFEAT = 4096

_K = 16
_CH = _CAPACITY // _K


def _body(f_hbm, m_hbm, o_hbm, fvmem, rowbuf, copy_sems, f_sem, row_sem):
    def chunk_copy(k):
        return pltpu.make_async_copy(
            m_hbm.at[pl.ds(k * _CH, _CH), :],
            o_hbm.at[pl.ds(k * _CH, _CH), :],
            copy_sems.at[k],
        )

    for k in range(_K):
        chunk_copy(k).start()
    fcopy = pltpu.make_async_copy(f_hbm, fvmem, f_sem)
    fcopy.start()
    fcopy.wait()
    rowbuf[...] = jnp.sum(fvmem[...], axis=0, keepdims=True) * (1.0 / _NFEAT)

    chunk_copy(_PTR // _CH).wait()
    rcopy = pltpu.make_async_copy(rowbuf, o_hbm.at[pl.ds(_PTR, 1), :], row_sem)
    rcopy.start()
    rcopy.wait()
    for k in range(_K):
        if k != _PTR // _CH:
            chunk_copy(k).wait()


def kernel(feature, memory):
    return pl.pallas_call(
        _body,
        in_specs=[
            pl.BlockSpec(memory_space=pl.ANY),
            pl.BlockSpec(memory_space=pl.ANY),
        ],
        out_specs=pl.BlockSpec(memory_space=pl.ANY),
        out_shape=jax.ShapeDtypeStruct((_CAPACITY, _EMBED), jnp.float32),
        scratch_shapes=[
            pltpu.VMEM((_NFEAT, _EMBED), jnp.float32),
            pltpu.VMEM((1, _EMBED), jnp.float32),
            pltpu.SemaphoreType.DMA((_K,)),
            pltpu.SemaphoreType.DMA,
            pltpu.SemaphoreType.DMA,
        ],
    )(feature, memory)


# speedup vs baseline: 40.9550x; 40.9550x over previous
import jax
import jax.numpy as jnp
from jax.experimental import pallas as pl
from jax.experimental.pallas import tpu as pltpu

_CAPACITY = 65536
_EMBED = 256
_PTR = 0
_NFEAT = 4096

_NCH = 32
_CROWS = _CAPACITY // _NCH
_NBUF = 4


def _body(f_hbm, m_hbm, o_hbm, fvmem, bufs, rowbuf,
          in_sems, out_sems, f_sem, row_sem):
    def in_copy(i):
        return pltpu.make_async_copy(
            m_hbm.at[pl.ds(i * _CROWS, _CROWS), :],
            bufs.at[i % _NBUF],
            in_sems.at[i % _NBUF],
        )

    def out_copy(i):
        return pltpu.make_async_copy(
            bufs.at[i % _NBUF],
            o_hbm.at[pl.ds(i * _CROWS, _CROWS), :],
            out_sems.at[i % _NBUF],
        )

    fcopy = pltpu.make_async_copy(f_hbm, fvmem, f_sem)
    fcopy.start()
    for b in range(_NBUF):
        in_copy(b).start()
    fcopy.wait()
    rowbuf[...] = jnp.sum(fvmem[...], axis=0, keepdims=True) * (1.0 / _NFEAT)

    patch = pltpu.make_async_copy(rowbuf, o_hbm.at[pl.ds(_PTR, 1), :], row_sem)
    for i in range(_NCH):
        in_copy(i).wait()
        out_copy(i).start()
        nxt = i + _NBUF
        if nxt < _NCH:
            out_copy(i).wait()
            in_copy(nxt).start()
        if i == _PTR // _CROWS:
            patch.start()
    for i in range(_NCH - _NBUF, _NCH):
        out_copy(i).wait()
    patch.wait()


def kernel(feature, memory):
    return pl.pallas_call(
        _body,
        in_specs=[
            pl.BlockSpec(memory_space=pl.ANY),
            pl.BlockSpec(memory_space=pl.ANY),
        ],
        out_specs=pl.BlockSpec(memory_space=pl.ANY),
        out_shape=jax.ShapeDtypeStruct((_CAPACITY, _EMBED), jnp.float32),
        scratch_shapes=[
            pltpu.VMEM((_NFEAT, _EMBED), jnp.float32),
            pltpu.VMEM((_NBUF, _CROWS, _EMBED), jnp.float32),
            pltpu.VMEM((1, _EMBED), jnp.float32),
            pltpu.SemaphoreType.DMA((_NBUF,)),
            pltpu.SemaphoreType.DMA((_NBUF,)),
            pltpu.SemaphoreType.DMA,
            pltpu.SemaphoreType.DMA,
        ],
    )(feature, memory)


# speedup vs baseline: 46.8163x; 1.1431x over previous
import jax
import jax.numpy as jnp
from jax.experimental import pallas as pl
from jax.experimental.pallas import tpu as pltpu

_CAPACITY = 65536
_EMBED = 256
_PTR = 0
_NFEAT = 4096

_NCH = 32
_CROWS = _CAPACITY // _NCH
_NBUF = 8
_SLACK = 3


def _body(f_hbm, m_hbm, o_hbm, fvmem, bufs, rowbuf,
          in_sems, out_sems, f_sem, row_sem):
    def in_copy(i):
        return pltpu.make_async_copy(
            m_hbm.at[pl.ds(i * _CROWS, _CROWS), :],
            bufs.at[i % _NBUF],
            in_sems.at[i % _NBUF],
        )

    def out_copy(i):
        return pltpu.make_async_copy(
            bufs.at[i % _NBUF],
            o_hbm.at[pl.ds(i * _CROWS, _CROWS), :],
            out_sems.at[i % _NBUF],
        )

    fcopy = pltpu.make_async_copy(f_hbm, fvmem, f_sem)
    fcopy.start()
    for b in range(_NBUF):
        in_copy(b).start()
    fcopy.wait()
    rowbuf[...] = jnp.sum(fvmem[...], axis=0, keepdims=True) * (1.0 / _NFEAT)

    patch = pltpu.make_async_copy(rowbuf, o_hbm.at[pl.ds(_PTR, 1), :], row_sem)
    for i in range(_NCH):
        in_copy(i).wait()
        out_copy(i).start()
        j = i - _SLACK
        if j >= 0:
            out_copy(j).wait()
            if j + _NBUF < _NCH:
                in_copy(j + _NBUF).start()
            if j == _PTR // _CROWS:
                patch.start()
    for j in range(max(0, _NCH - _SLACK), _NCH):
        out_copy(j).wait()
    patch.wait()


def kernel(feature, memory):
    return pl.pallas_call(
        _body,
        in_specs=[
            pl.BlockSpec(memory_space=pl.ANY),
            pl.BlockSpec(memory_space=pl.ANY),
        ],
        out_specs=pl.BlockSpec(memory_space=pl.ANY),
        out_shape=jax.ShapeDtypeStruct((_CAPACITY, _EMBED), jnp.float32),
        scratch_shapes=[
            pltpu.VMEM((_NFEAT, _EMBED), jnp.float32),
            pltpu.VMEM((_NBUF, _CROWS, _EMBED), jnp.float32),
            pltpu.VMEM((1, _EMBED), jnp.float32),
            pltpu.SemaphoreType.DMA((_NBUF,)),
            pltpu.SemaphoreType.DMA((_NBUF,)),
            pltpu.SemaphoreType.DMA,
            pltpu.SemaphoreType.DMA,
        ],
    )(feature, memory)


# speedup vs baseline: 47.8865x; 1.0229x over previous
import jax
import jax.numpy as jnp
from jax.experimental import pallas as pl
from jax.experimental.pallas import tpu as pltpu

_CAPACITY = 65536
_EMBED = 256
_PTR = 0
_NFEAT = 4096

_NCH = 16
_CROWS = _CAPACITY // _NCH
_NBUF = 6
_SLACK = 2


def _body(f_hbm, m_hbm, o_hbm, fvmem, bufs, rowbuf,
          in_sems, out_sems, f_sem, row_sem):
    def in_copy(i):
        return pltpu.make_async_copy(
            m_hbm.at[pl.ds(i * _CROWS, _CROWS), :],
            bufs.at[i % _NBUF],
            in_sems.at[i % _NBUF],
        )

    def out_copy(i):
        return pltpu.make_async_copy(
            bufs.at[i % _NBUF],
            o_hbm.at[pl.ds(i * _CROWS, _CROWS), :],
            out_sems.at[i % _NBUF],
        )

    fcopy = pltpu.make_async_copy(f_hbm, fvmem, f_sem)
    fcopy.start()
    for b in range(_NBUF):
        in_copy(b).start()
    fcopy.wait()
    rowbuf[...] = jnp.sum(fvmem[...], axis=0, keepdims=True) * (1.0 / _NFEAT)

    patch = pltpu.make_async_copy(rowbuf, o_hbm.at[pl.ds(_PTR, 1), :], row_sem)
    for i in range(_NCH):
        in_copy(i).wait()
        out_copy(i).start()
        j = i - _SLACK
        if j >= 0:
            out_copy(j).wait()
            if j + _NBUF < _NCH:
                in_copy(j + _NBUF).start()
            if j == _PTR // _CROWS:
                patch.start()
    for j in range(max(0, _NCH - _SLACK), _NCH):
        out_copy(j).wait()
    patch.wait()


def kernel(feature, memory):
    return pl.pallas_call(
        _body,
        in_specs=[
            pl.BlockSpec(memory_space=pl.ANY),
            pl.BlockSpec(memory_space=pl.ANY),
        ],
        out_specs=pl.BlockSpec(memory_space=pl.ANY),
        out_shape=jax.ShapeDtypeStruct((_CAPACITY, _EMBED), jnp.float32),
        scratch_shapes=[
            pltpu.VMEM((_NFEAT, _EMBED), jnp.float32),
            pltpu.VMEM((_NBUF, _CROWS, _EMBED), jnp.float32),
            pltpu.VMEM((1, _EMBED), jnp.float32),
            pltpu.SemaphoreType.DMA((_NBUF,)),
            pltpu.SemaphoreType.DMA((_NBUF,)),
            pltpu.SemaphoreType.DMA,
            pltpu.SemaphoreType.DMA,
        ],
    )(feature, memory)


# speedup vs baseline: 49.2429x; 1.0283x over previous
import jax
import jax.numpy as jnp
from jax.experimental import pallas as pl
from jax.experimental.pallas import tpu as pltpu

_CAPACITY = 65536
_EMBED = 256
_PTR = 0
_NFEAT = 4096

_NCH = 8
_CROWS = _CAPACITY // _NCH
_NBUF = 5
_SLACK = 2


def _body(f_hbm, m_hbm, o_hbm, fvmem, bufs, rowbuf,
          in_sems, out_sems, f_sem, row_sem):
    def in_copy(i):
        return pltpu.make_async_copy(
            m_hbm.at[pl.ds(i * _CROWS, _CROWS), :],
            bufs.at[i % _NBUF],
            in_sems.at[i % _NBUF],
        )

    def out_copy(i):
        return pltpu.make_async_copy(
            bufs.at[i % _NBUF],
            o_hbm.at[pl.ds(i * _CROWS, _CROWS), :],
            out_sems.at[i % _NBUF],
        )

    fcopy = pltpu.make_async_copy(f_hbm, fvmem, f_sem)
    fcopy.start()
    for b in range(_NBUF):
        in_copy(b).start()
    fcopy.wait()
    rowbuf[...] = jnp.sum(fvmem[...], axis=0, keepdims=True) * (1.0 / _NFEAT)

    patch = pltpu.make_async_copy(rowbuf, o_hbm.at[pl.ds(_PTR, 1), :], row_sem)
    for i in range(_NCH):
        in_copy(i).wait()
        out_copy(i).start()
        j = i - _SLACK
        if j >= 0:
            out_copy(j).wait()
            if j + _NBUF < _NCH:
                in_copy(j + _NBUF).start()
            if j == _PTR // _CROWS:
                patch.start()
    for j in range(max(0, _NCH - _SLACK), _NCH):
        out_copy(j).wait()
    patch.wait()


def kernel(feature, memory):
    return pl.pallas_call(
        _body,
        in_specs=[
            pl.BlockSpec(memory_space=pl.ANY),
            pl.BlockSpec(memory_space=pl.ANY),
        ],
        out_specs=pl.BlockSpec(memory_space=pl.ANY),
        out_shape=jax.ShapeDtypeStruct((_CAPACITY, _EMBED), jnp.float32),
        scratch_shapes=[
            pltpu.VMEM((_NFEAT, _EMBED), jnp.float32),
            pltpu.VMEM((_NBUF, _CROWS, _EMBED), jnp.float32),
            pltpu.VMEM((1, _EMBED), jnp.float32),
            pltpu.SemaphoreType.DMA((_NBUF,)),
            pltpu.SemaphoreType.DMA((_NBUF,)),
            pltpu.SemaphoreType.DMA,
            pltpu.SemaphoreType.DMA,
        ],
    )(feature, memory)


# speedup vs baseline: 49.6221x; 1.0077x over previous
import jax
import jax.numpy as jnp
from jax.experimental import pallas as pl
from jax.experimental.pallas import tpu as pltpu

_CAPACITY = 65536
_EMBED = 256
_PTR = 0
_NFEAT = 4096

_NCH = 4
_CROWS = _CAPACITY // _NCH
_NBUF = 3
_SLACK = 1


def _body(f_hbm, m_hbm, o_hbm, fvmem, bufs, rowbuf,
          in_sems, out_sems, f_sem, row_sem):
    def in_copy(i):
        return pltpu.make_async_copy(
            m_hbm.at[pl.ds(i * _CROWS, _CROWS), :],
            bufs.at[i % _NBUF],
            in_sems.at[i % _NBUF],
        )

    def out_copy(i):
        return pltpu.make_async_copy(
            bufs.at[i % _NBUF],
            o_hbm.at[pl.ds(i * _CROWS, _CROWS), :],
            out_sems.at[i % _NBUF],
        )

    fcopy = pltpu.make_async_copy(f_hbm, fvmem, f_sem)
    fcopy.start()
    for b in range(_NBUF):
        in_copy(b).start()
    fcopy.wait()
    rowbuf[...] = jnp.sum(fvmem[...], axis=0, keepdims=True) * (1.0 / _NFEAT)

    patch = pltpu.make_async_copy(rowbuf, o_hbm.at[pl.ds(_PTR, 1), :], row_sem)
    for i in range(_NCH):
        in_copy(i).wait()
        out_copy(i).start()
        j = i - _SLACK
        if j >= 0:
            out_copy(j).wait()
            if j + _NBUF < _NCH:
                in_copy(j + _NBUF).start()
            if j == _PTR // _CROWS:
                patch.start()
    for j in range(max(0, _NCH - _SLACK), _NCH):
        out_copy(j).wait()
    patch.wait()


def kernel(feature, memory):
    return pl.pallas_call(
        _body,
        in_specs=[
            pl.BlockSpec(memory_space=pl.ANY),
            pl.BlockSpec(memory_space=pl.ANY),
        ],
        out_specs=pl.BlockSpec(memory_space=pl.ANY),
        out_shape=jax.ShapeDtypeStruct((_CAPACITY, _EMBED), jnp.float32),
        scratch_shapes=[
            pltpu.VMEM((_NFEAT, _EMBED), jnp.float32),
            pltpu.VMEM((_NBUF, _CROWS, _EMBED), jnp.float32),
            pltpu.VMEM((1, _EMBED), jnp.float32),
            pltpu.SemaphoreType.DMA((_NBUF,)),
            pltpu.SemaphoreType.DMA((_NBUF,)),
            pltpu.SemaphoreType.DMA,
            pltpu.SemaphoreType.DMA,
        ],
    )(feature, memory)
